# baseline (speedup 1.0000x reference)
import jax
import jax.numpy as jnp
from jax import lax
from jax.experimental import pallas as pl
from jax.experimental.pallas import tpu as pltpu

_HIGHEST = jax.lax.Precision.HIGHEST
_BLOCK_B = 16384


def _wl_body(a_ref, w_ref, b_ref, x_ref, o_ref):
    T = x_ref.shape[0]
    m = jnp.dot(a_ref[...], w_ref[...],
                preferred_element_type=jnp.float32, precision=_HIGHEST)
    for t in range(T):
        y = lax.dot_general(m, x_ref[t], (((0,), (0,)), ((), ())),
                            preferred_element_type=jnp.float32)
        bt = jnp.broadcast_to(b_ref[:, t:t + 1], y.shape)
        o_ref[t] = jnp.maximum(y + bt, 0.0).astype(o_ref.dtype)


def kernel(x, corr_adj, w, bias):
    B, T, C = x.shape
    dtype = x.dtype

    bias_ct = (jnp.zeros((C, T), dtype) if bias is None
               else jnp.transpose(bias, (1, 0)))
    x_t = jnp.transpose(x, (1, 2, 0))
    blk = min(_BLOCK_B, B)
    grid = (pl.cdiv(B, blk),)

    out_t = pl.pallas_call(
        _wl_body,
        out_shape=jax.ShapeDtypeStruct((T, C, B), dtype),
        grid=grid,
        in_specs=[
            pl.BlockSpec((C, C), lambda i: (0, 0)),
            pl.BlockSpec((C, C), lambda i: (0, 0)),
            pl.BlockSpec((C, T), lambda i: (0, 0)),
            pl.BlockSpec((T, C, blk), lambda i: (0, 0, i)),
        ],
        out_specs=pl.BlockSpec((T, C, blk), lambda i: (0, 0, i)),
        compiler_params=pltpu.CompilerParams(
            dimension_semantics=("parallel",)),
    )(corr_adj, w, bias_ct, x_t)

    return jnp.transpose(out_t, (2, 0, 1))


# speedup vs baseline: 1.0336x; 1.0336x over previous
import jax
import jax.numpy as jnp
from jax import lax
from jax.experimental import pallas as pl
from jax.experimental.pallas import tpu as pltpu

_HIGHEST = jax.lax.Precision.HIGHEST
_BLOCK_B = 32768


def _wl_body(a_ref, w_ref, b_ref, x_ref, o_ref):
    T = x_ref.shape[0]
    m = jnp.dot(a_ref[...], w_ref[...],
                preferred_element_type=jnp.float32, precision=_HIGHEST)
    for t in range(T):
        y = lax.dot_general(m, x_ref[t], (((0,), (0,)), ((), ())),
                            preferred_element_type=jnp.float32)
        bt = jnp.broadcast_to(b_ref[:, t:t + 1], y.shape)
        o_ref[t] = jnp.maximum(y + bt, 0.0).astype(o_ref.dtype)


def kernel(x, corr_adj, w, bias):
    B, T, C = x.shape
    dtype = x.dtype

    bias_ct = (jnp.zeros((C, T), dtype) if bias is None
               else jnp.transpose(bias, (1, 0)))
    x_t = jnp.transpose(x, (1, 2, 0))
    blk = min(_BLOCK_B, B)
    grid = (pl.cdiv(B, blk),)

    out_t = pl.pallas_call(
        _wl_body,
        out_shape=jax.ShapeDtypeStruct((T, C, B), dtype),
        grid=grid,
        in_specs=[
            pl.BlockSpec((C, C), lambda i: (0, 0)),
            pl.BlockSpec((C, C), lambda i: (0, 0)),
            pl.BlockSpec((C, T), lambda i: (0, 0)),
            pl.BlockSpec((T, C, blk), lambda i: (0, 0, i)),
        ],
        out_specs=pl.BlockSpec((T, C, blk), lambda i: (0, 0, i)),
        compiler_params=pltpu.CompilerParams(
            dimension_semantics=("parallel",)),
    )(corr_adj, w, bias_ct, x_t)

    return jnp.transpose(out_t, (2, 0, 1))
